# baseline (speedup 1.0000x reference)
import functools

import jax
import jax.numpy as jnp
from jax.experimental import pallas as pl
from jax.experimental.pallas import tpu as pltpu

RB = 32
CH = 2176
NEG = -1.7e38


def _body(x_ref, w0_ref, b0_ref, w1_ref, b1_ref, w2_ref, b2_ref, o_ref, *, K):
    V = x_ref.shape[1]
    n_full = V // CH
    rem = V - n_full * CH
    W = n_full + (1 if rem else 0)

    chunks = [x_ref[:, w * CH:(w + 1) * CH] for w in range(n_full)]
    if rem:
        tail = jnp.concatenate(
            [x_ref[:, n_full * CH:],
             jnp.full((RB, CH - rem), NEG, jnp.float32)], axis=1)
        chunks.append(tail)
    x3 = jnp.stack(chunks, axis=1)
    gm = jnp.max(x3, axis=1)

    giota = jax.lax.broadcasted_iota(jnp.int32, gm.shape, 1)
    ids = []
    g = gm
    for _ in range(K):
        m = jnp.max(g, axis=1, keepdims=True)
        i = jnp.min(jnp.where(g == m, giota, CH), axis=1, keepdims=True)
        ids.append(i)
        g = jnp.where(giota == i, NEG, g)
    idx = jnp.concatenate(ids, axis=1)

    oh = (jax.lax.broadcasted_iota(jnp.int32, (RB, K, CH), 2)
          == idx[:, :, None]).astype(jnp.float32)
    cand = jax.lax.dot_general(
        oh, x3, (((2,), (2,)), ((0,), (0,))),
        precision=jax.lax.Precision.HIGHEST,
        preferred_element_type=jnp.float32)
    c = cand.reshape(RB, K * W)

    ncand = K * W
    cid = jax.lax.broadcasted_iota(jnp.int32, c.shape, 1)
    vals = []
    for _ in range(K):
        m = jnp.max(c, axis=1, keepdims=True)
        i = jnp.min(jnp.where(c == m, cid, ncand), axis=1, keepdims=True)
        c = jnp.where(cid == i, NEG, c)
        vals.append(m)
    topk = jnp.concatenate(vals, axis=1)

    bf = jnp.bfloat16
    h = jnp.maximum(
        jax.lax.dot_general(topk.astype(bf), w0_ref[...].astype(bf),
                            (((1,), (0,)), ((), ())),
                            preferred_element_type=jnp.float32)
        + b0_ref[...], 0.0)
    h = jnp.maximum(
        jax.lax.dot_general(h.astype(bf), w1_ref[...].astype(bf),
                            (((1,), (0,)), ((), ())),
                            preferred_element_type=jnp.float32)
        + b1_ref[...], 0.0)
    t = jnp.sum(h.astype(bf).astype(jnp.float32)
                * w2_ref[...].astype(bf).astype(jnp.float32),
                axis=1, keepdims=True) + b2_ref[...]
    t = jnp.clip(jnp.abs(t), 1e-12, 1e12)

    mrow = topk[:, 0:1]
    s = jnp.zeros((RB, 1), jnp.float32)
    for w in range(W):
        lo = w * CH
        hi = min((w + 1) * CH, V)
        e = jnp.exp((x_ref[:, lo:hi] - mrow) / t)
        s = s + jnp.sum(e, axis=1, keepdims=True)
        o_ref[:, lo:hi] = e
    rs = 1.0 / s
    for w in range(W):
        lo = w * CH
        hi = min((w + 1) * CH, V)
        o_ref[:, lo:hi] = o_ref[:, lo:hi] * rs


def kernel(inp, W0, b0, W1, b1, W2, b2):
    B, V = inp.shape
    K, H = W0.shape
    grid = (B // RB,)
    out = pl.pallas_call(
        functools.partial(_body, K=K),
        grid=grid,
        in_specs=[
            pl.BlockSpec((RB, V), lambda i: (i, 0)),
            pl.BlockSpec((K, H), lambda i: (0, 0)),
            pl.BlockSpec((1, H), lambda i: (0, 0)),
            pl.BlockSpec((H, H), lambda i: (0, 0)),
            pl.BlockSpec((1, H), lambda i: (0, 0)),
            pl.BlockSpec((1, H), lambda i: (0, 0)),
            pl.BlockSpec((1, 1), lambda i: (0, 0)),
        ],
        out_specs=pl.BlockSpec((RB, V), lambda i: (i, 0)),
        out_shape=jax.ShapeDtypeStruct((B, V), jnp.float32),
        compiler_params=pltpu.CompilerParams(
            dimension_semantics=("arbitrary",),
            vmem_limit_bytes=100 * 1024 * 1024,
        ),
    )(inp, W0, b0.reshape(1, H), W1, b1.reshape(1, H),
      W2.reshape(1, H), b2.reshape(1, 1))
    return out


# speedup vs baseline: 1.3888x; 1.3888x over previous
import functools

import jax
import jax.numpy as jnp
from jax.experimental import pallas as pl
from jax.experimental.pallas import tpu as pltpu

RB = 32
CH = 2176
NEG = -1.7e38


def _body(x_ref, w0_ref, b0_ref, w1_ref, b1_ref, w2_ref, b2_ref, o_ref, *, K):
    V = x_ref.shape[1]
    n_full = V // CH
    rem = V - n_full * CH
    W = n_full + (1 if rem else 0)

    chunks = [x_ref[:, w * CH:(w + 1) * CH] for w in range(n_full)]
    if rem:
        tail = jnp.concatenate(
            [x_ref[:, n_full * CH:],
             jnp.full((RB, CH - rem), NEG, jnp.float32)], axis=1)
        chunks.append(tail)
    x3 = jnp.stack(chunks, axis=1)
    gm = jnp.max(x3, axis=1)

    giota = jax.lax.broadcasted_iota(jnp.int32, gm.shape, 1)
    ids = []
    g = gm
    for _ in range(K):
        m = jnp.max(g, axis=1, keepdims=True)
        i = jnp.min(jnp.where(g == m, giota, CH), axis=1, keepdims=True)
        ids.append(i)
        g = jnp.where(giota == i, NEG, g)
    idx = jnp.concatenate(ids, axis=1)

    oh = (jax.lax.broadcasted_iota(jnp.int32, (RB, K, CH), 2)
          == idx[:, :, None]).astype(jnp.float32)
    cand = jax.lax.dot_general(
        oh, x3, (((2,), (2,)), ((0,), (0,))),
        preferred_element_type=jnp.float32)
    c = cand.reshape(RB, K * W)

    ncand = K * W
    cid = jax.lax.broadcasted_iota(jnp.int32, c.shape, 1)
    vals = []
    for _ in range(K):
        m = jnp.max(c, axis=1, keepdims=True)
        i = jnp.min(jnp.where(c == m, cid, ncand), axis=1, keepdims=True)
        c = jnp.where(cid == i, NEG, c)
        vals.append(m)
    topk = jnp.concatenate(vals, axis=1)

    bf = jnp.bfloat16
    h = jnp.maximum(
        jax.lax.dot_general(topk.astype(bf), w0_ref[...].astype(bf),
                            (((1,), (0,)), ((), ())),
                            preferred_element_type=jnp.float32)
        + b0_ref[...], 0.0)
    h = jnp.maximum(
        jax.lax.dot_general(h.astype(bf), w1_ref[...].astype(bf),
                            (((1,), (0,)), ((), ())),
                            preferred_element_type=jnp.float32)
        + b1_ref[...], 0.0)
    t = jnp.sum(h.astype(bf).astype(jnp.float32)
                * w2_ref[...].astype(bf).astype(jnp.float32),
                axis=1, keepdims=True) + b2_ref[...]
    t = jnp.clip(jnp.abs(t), 1e-12, 1e12)

    mrow = topk[:, 0:1]
    rt = 1.0 / t
    s = jnp.zeros((RB, 1), jnp.float32)
    for w in range(W):
        lo = w * CH
        hi = min((w + 1) * CH, V)
        e = jnp.exp((x_ref[:, lo:hi] - mrow) * rt)
        s = s + jnp.sum(e, axis=1, keepdims=True)
        o_ref[:, lo:hi] = e
    rs = 1.0 / s
    for w in range(W):
        lo = w * CH
        hi = min((w + 1) * CH, V)
        o_ref[:, lo:hi] = o_ref[:, lo:hi] * rs


def kernel(inp, W0, b0, W1, b1, W2, b2):
    B, V = inp.shape
    K, H = W0.shape
    grid = (B // RB,)
    out = pl.pallas_call(
        functools.partial(_body, K=K),
        grid=grid,
        in_specs=[
            pl.BlockSpec((RB, V), lambda i: (i, 0)),
            pl.BlockSpec((K, H), lambda i: (0, 0)),
            pl.BlockSpec((1, H), lambda i: (0, 0)),
            pl.BlockSpec((H, H), lambda i: (0, 0)),
            pl.BlockSpec((1, H), lambda i: (0, 0)),
            pl.BlockSpec((1, H), lambda i: (0, 0)),
            pl.BlockSpec((1, 1), lambda i: (0, 0)),
        ],
        out_specs=pl.BlockSpec((RB, V), lambda i: (i, 0)),
        out_shape=jax.ShapeDtypeStruct((B, V), jnp.float32),
        compiler_params=pltpu.CompilerParams(
            dimension_semantics=("arbitrary",),
            vmem_limit_bytes=100 * 1024 * 1024,
        ),
    )(inp, W0, b0.reshape(1, H), W1, b1.reshape(1, H),
      W2.reshape(1, H), b2.reshape(1, 1))
    return out


# speedup vs baseline: 1.7566x; 1.2648x over previous
import functools

import jax
import jax.numpy as jnp
from jax.experimental import pallas as pl
from jax.experimental.pallas import tpu as pltpu

RB = 32
CH = 2176
NEG = -1.7e38


def _body(x_ref, w0_ref, b0_ref, w1_ref, b1_ref, w2_ref, b2_ref, o_ref, *, K):
    V = x_ref.shape[1]
    n_full = V // CH
    rem = V - n_full * CH
    W = n_full + (1 if rem else 0)

    xp = jnp.concatenate(
        [x_ref[...], jnp.full((RB, W * CH - V), NEG, jnp.float32)], axis=1)
    x3 = xp.reshape(RB, W, CH)
    gm = jnp.max(x3, axis=1)

    giota = jax.lax.broadcasted_iota(jnp.int32, gm.shape, 1)
    ids = []
    g = gm
    for _ in range(K):
        i = jnp.argmax(g, axis=1).astype(jnp.int32)[:, None]
        ids.append(i)
        g = jnp.where(giota == i, NEG, g)
    idx = jnp.concatenate(ids, axis=1)

    oh = (jax.lax.broadcasted_iota(jnp.int32, (RB, K, CH), 2)
          == idx[:, :, None]).astype(jnp.float32)
    cand = jax.lax.dot_general(
        oh, x3, (((2,), (2,)), ((0,), (0,))),
        preferred_element_type=jnp.float32)
    c = cand.reshape(RB, K * W)

    ncand = K * W
    cid = jax.lax.broadcasted_iota(jnp.int32, c.shape, 1)
    vals = []
    for _ in range(K):
        m = jnp.max(c, axis=1, keepdims=True)
        i = jnp.argmax(c, axis=1).astype(jnp.int32)[:, None]
        c = jnp.where(cid == i, NEG, c)
        vals.append(m)
    topk = jnp.concatenate(vals, axis=1)

    bf = jnp.bfloat16
    h = jnp.maximum(
        jax.lax.dot_general(topk.astype(bf), w0_ref[...].astype(bf),
                            (((1,), (0,)), ((), ())),
                            preferred_element_type=jnp.float32)
        + b0_ref[...], 0.0)
    h = jnp.maximum(
        jax.lax.dot_general(h.astype(bf), w1_ref[...].astype(bf),
                            (((1,), (0,)), ((), ())),
                            preferred_element_type=jnp.float32)
        + b1_ref[...], 0.0)
    t = jnp.sum(h.astype(bf).astype(jnp.float32)
                * w2_ref[...].astype(bf).astype(jnp.float32),
                axis=1, keepdims=True) + b2_ref[...]
    t = jnp.clip(jnp.abs(t), 1e-12, 1e12)

    mrow = topk[:, 0:1]
    rt = 1.0 / t
    s = jnp.zeros((RB, 1), jnp.float32)
    for w in range(W):
        lo = w * CH
        hi = min((w + 1) * CH, V)
        e = jnp.exp((x_ref[:, lo:hi] - mrow) * rt)
        s = s + jnp.sum(e, axis=1, keepdims=True)
        o_ref[:, lo:hi] = e
    rs = 1.0 / s
    for w in range(W):
        lo = w * CH
        hi = min((w + 1) * CH, V)
        o_ref[:, lo:hi] = o_ref[:, lo:hi] * rs


def kernel(inp, W0, b0, W1, b1, W2, b2):
    B, V = inp.shape
    K, H = W0.shape
    grid = (B // RB,)
    out = pl.pallas_call(
        functools.partial(_body, K=K),
        grid=grid,
        in_specs=[
            pl.BlockSpec((RB, V), lambda i: (i, 0)),
            pl.BlockSpec((K, H), lambda i: (0, 0)),
            pl.BlockSpec((1, H), lambda i: (0, 0)),
            pl.BlockSpec((H, H), lambda i: (0, 0)),
            pl.BlockSpec((1, H), lambda i: (0, 0)),
            pl.BlockSpec((1, H), lambda i: (0, 0)),
            pl.BlockSpec((1, 1), lambda i: (0, 0)),
        ],
        out_specs=pl.BlockSpec((RB, V), lambda i: (i, 0)),
        out_shape=jax.ShapeDtypeStruct((B, V), jnp.float32),
        compiler_params=pltpu.CompilerParams(
            dimension_semantics=("arbitrary",),
            vmem_limit_bytes=100 * 1024 * 1024,
        ),
    )(inp, W0, b0.reshape(1, H), W1, b1.reshape(1, H),
      W2.reshape(1, H), b2.reshape(1, 1))
    return out
